# baseline (speedup 1.0000x reference)
import functools

import jax
import jax.numpy as jnp
from jax import lax
from jax.experimental import pallas as pl
from jax.experimental.pallas import tpu as pltpu
from jax.experimental.pallas import tpu_sc as plsc

_N = 10000
_E = 320000
_H = 8
_D = 16
_NEG_SLOPE = 0.2
_HD = _H * _D
_W = 144

_NC = 2
_NS = 16
_NW = _NC * _NS
_EPT = _E // _NW
_C = 80
_ITERS = _EPT // _C
_RPT8 = 624
_TAIL = _N - _NS * _RPT8


def _prep_body(hs_ref, attn_ref, gsel_ref, gselt_ref, out_ref):
    hs = hs_ref[...]
    attn = attn_ref[...]
    gsel = gsel_ref[...]
    gselt = gselt_ref[...]
    el = jnp.dot(hs * attn, gsel, precision=lax.Precision.HIGHEST)
    el = jnp.where(el > 0, el, _NEG_SLOPE * el)
    m = jnp.max(el)
    q = jnp.exp(el - m)
    qb = jnp.dot(q, gselt, precision=lax.Precision.HIGHEST)
    w = hs * qb
    pad = jnp.zeros((hs.shape[0], _W - _HD - _H), jnp.float32)
    out_ref[...] = jnp.concatenate([w, q, pad], axis=1)


def _prep(h_src, attn_flat, gsel, gselt):
    return pl.pallas_call(
        _prep_body,
        out_shape=jax.ShapeDtypeStruct((_N, _W), jnp.float32),
    )(h_src, attn_flat, gsel, gselt)


def _edge_body(w_hbm, ei_hbm, out_hbm,
               sidx, didx0, didx1, rows0, rows1, acc,
               gsem0, gsem1, isem0, isem1):
    c = lax.axis_index("c")
    s = lax.axis_index("s")
    wid = s * _NC + c
    ebase = pl.multiple_of(wid * _EPT, 8)

    pltpu.async_copy(ei_hbm.at[0, pl.ds(ebase, _EPT)], sidx, isem0)

    @pl.loop(0, _C)
    def _zrow(r):
        for k in range(_W // 16):
            rows0[r, pl.ds(16 * k, 16)] = jnp.zeros((16,), jnp.float32)

    @pl.loop(0, 7)
    def _zcp(j):
        zoff = pl.multiple_of(s * _RPT8 + j * _C, 8)
        pltpu.sync_copy(rows0, acc.at[pl.ds(zoff, _C)])

    zoff7 = pl.multiple_of(s * _RPT8 + 7 * _C, 8)
    pltpu.sync_copy(rows0.at[pl.ds(0, _RPT8 - 7 * _C)],
                    acc.at[pl.ds(zoff7, _RPT8 - 7 * _C)])

    @pl.when(s == _NS - 1)
    def _zero_tail():
        pltpu.sync_copy(rows0.at[pl.ds(0, _TAIL)],
                        acc.at[pl.ds(_NS * _RPT8, _TAIL)])

    pltpu.make_async_copy(ei_hbm.at[0, pl.ds(ebase, _EPT)], sidx, isem0).wait()
    plsc.subcore_barrier()

    def _dst_chunk(i):
        off = pl.multiple_of(ebase + i * _C, 8)
        return ei_hbm.at[1, pl.ds(off, _C)]

    def _gather(i, rows, gsem):
        pltpu.async_copy(w_hbm.at[sidx.at[pl.ds(i * _C, _C)]], rows, gsem)

    def _gwait(rows, gsem):
        pltpu.make_async_copy(w_hbm.at[sidx.at[pl.ds(0, _C)]], rows,
                              gsem).wait()

    pltpu.async_copy(_dst_chunk(0), didx0, isem0)
    pltpu.async_copy(_dst_chunk(1), didx1, isem1)
    _gather(0, rows0, gsem0)
    pltpu.make_async_copy(_dst_chunk(0), didx0, isem0).wait()

    @pl.loop(0, _ITERS - 1, step=2)
    def _pair(g):
        _gwait(rows0, gsem0)
        _gather(g + 1, rows1, gsem1)
        pltpu.sync_copy(rows0, acc.at[didx0], add=True)

        @pl.when(g + 2 < _ITERS)
        def _idx2():
            pltpu.async_copy(_dst_chunk(g + 2), didx0, isem0)

        _gwait(rows1, gsem1)
        pltpu.make_async_copy(_dst_chunk(g + 1), didx1, isem1).wait()

        @pl.when(g + 2 < _ITERS)
        def _g2():
            pltpu.make_async_copy(_dst_chunk(g + 2), didx0, isem0).wait()
            _gather(g + 2, rows0, gsem0)

        pltpu.sync_copy(rows1, acc.at[didx1], add=True)

        @pl.when(g + 3 < _ITERS)
        def _idx3():
            pltpu.async_copy(_dst_chunk(g + 3), didx1, isem1)

    _gwait(rows0, gsem0)
    pltpu.sync_copy(rows0, acc.at[didx0], add=True)

    plsc.subcore_barrier()

    doff = pl.multiple_of(s * _RPT8, 8)
    pltpu.sync_copy(acc.at[pl.ds(doff, _RPT8)],
                    out_hbm.at[c, pl.ds(doff, _RPT8)])

    @pl.when(s == _NS - 1)
    def _drain_tail():
        pltpu.sync_copy(acc.at[pl.ds(_NS * _RPT8, _TAIL)],
                        out_hbm.at[c, pl.ds(_NS * _RPT8, _TAIL)])


@functools.partial(
    pl.kernel,
    mesh=plsc.VectorSubcoreMesh(core_axis_name="c", subcore_axis_name="s"),
    out_type=jax.ShapeDtypeStruct((_NC, _N, _W), jnp.float32),
    scratch_types=[
        pltpu.VMEM((_EPT,), jnp.int32),
        pltpu.VMEM((_C,), jnp.int32),
        pltpu.VMEM((_C,), jnp.int32),
        pltpu.VMEM((_C, _W), jnp.float32),
        pltpu.VMEM((_C, _W), jnp.float32),
        pltpu.VMEM_SHARED((_N, _W), jnp.float32),
        pltpu.SemaphoreType.DMA,
        pltpu.SemaphoreType.DMA,
        pltpu.SemaphoreType.DMA,
        pltpu.SemaphoreType.DMA,
    ],
    compiler_params=pltpu.CompilerParams(use_tc_tiling_on_sc=False),
)
def _edge_pass(w_hbm, ei_hbm, out_hbm,
               sidx, didx0, didx1, rows0, rows1, acc,
               gsem0, gsem1, isem0, isem1):
    _edge_body(w_hbm, ei_hbm, out_hbm,
               sidx, didx0, didx1, rows0, rows1, acc,
               gsem0, gsem1, isem0, isem1)


def _final_body(acc_ref, gselt_ref, out_ref):
    tot = acc_ref[0] + acc_ref[1]
    num = tot[:, 0:_HD]
    den = tot[:, _HD:_HD + _H]
    denb = jnp.dot(den, gselt_ref[...], precision=lax.Precision.HIGHEST)
    out_ref[...] = jnp.where(denb > 0, num / denb, 0.0)


def _final(acc, gselt):
    return pl.pallas_call(
        _final_body,
        out_shape=jax.ShapeDtypeStruct((_N, _HD), jnp.float32),
    )(acc, gselt)


def kernel(h_src, h_dst, attn_l, edge_index):
    del h_dst
    attn_flat = attn_l.reshape(1, _HD).astype(jnp.float32)
    gsel = (jnp.arange(_HD)[:, None] // _D == jnp.arange(_H)[None, :]
            ).astype(jnp.float32)
    gselt = gsel.T
    w = _prep(h_src, attn_flat, gsel, gselt)
    acc = _edge_pass(w, edge_index)
    return _final(acc, gselt)


# speedup vs baseline: 1.0057x; 1.0057x over previous
import functools

import jax
import jax.numpy as jnp
from jax import lax
from jax.experimental import pallas as pl
from jax.experimental.pallas import tpu as pltpu
from jax.experimental.pallas import tpu_sc as plsc

_N = 10000
_E = 320000
_H = 8
_D = 16
_NEG_SLOPE = 0.2
_HD = _H * _D
_W = 144

_NC = 2
_NS = 16
_NW = _NC * _NS
_EPT = _E // _NW
_C = 125
_ITERS = _EPT // _C
_RPT8 = 624
_TAIL = _N - _NS * _RPT8


def _prep_body(hs_ref, attn_ref, gsel_ref, gselt_ref, out_ref):
    hs = hs_ref[...]
    attn = attn_ref[...]
    gsel = gsel_ref[...]
    gselt = gselt_ref[...]
    el = jnp.dot(hs * attn, gsel, precision=lax.Precision.HIGHEST)
    el = jnp.where(el > 0, el, _NEG_SLOPE * el)
    m = jnp.max(el)
    q = jnp.exp(el - m)
    qb = jnp.dot(q, gselt, precision=lax.Precision.HIGHEST)
    w = hs * qb
    pad = jnp.zeros((hs.shape[0], _W - _HD - _H), jnp.float32)
    out_ref[...] = jnp.concatenate([w, q, pad], axis=1)


def _prep(h_src, attn_flat, gsel, gselt):
    return pl.pallas_call(
        _prep_body,
        out_shape=jax.ShapeDtypeStruct((_N, _W), jnp.float32),
    )(h_src, attn_flat, gsel, gselt)


def _edge_body(w_hbm, ei_hbm, out_hbm,
               sidx0, sidx1, didx0, didx1, rows0, rows1, acc,
               gsem0, gsem1, ssem0, ssem1, samA0, samA1, samB0, samB1):
    c = lax.axis_index("c")
    s = lax.axis_index("s")
    wid = s * _NC + c
    sbase = wid * _ITERS
    dbase = _E // _C + wid * _ITERS

    def _sidx(i, buf, sem):
        pltpu.async_copy(ei_hbm.at[sbase + i], buf, sem)

    def _sidx_wait(buf, sem):
        pltpu.make_async_copy(ei_hbm.at[sbase], buf, sem).wait()

    def _didx(i, buf, sem):
        pltpu.async_copy(ei_hbm.at[dbase + i], buf, sem)

    def _didx_wait(buf, sem):
        pltpu.make_async_copy(ei_hbm.at[dbase], buf, sem).wait()

    def _gather(sbuf, rows, gsem):
        pltpu.async_copy(w_hbm.at[sbuf], rows, gsem)

    def _gwait(rows, gsem):
        pltpu.make_async_copy(w_hbm.at[sidx0], rows, gsem).wait()

    _sidx(0, sidx0, samA0)
    _sidx(1, sidx1, samA1)
    _didx(0, didx0, samB0)
    _didx(1, didx1, samB1)

    @pl.loop(0, 120)
    def _zrow(r):
        for k in range(_W // 16):
            rows0[r, pl.ds(16 * k, 16)] = jnp.zeros((16,), jnp.float32)

    @pl.loop(0, 5)
    def _zcp(j):
        zoff = pl.multiple_of(s * _RPT8 + j * 120, 8)
        pltpu.sync_copy(rows0.at[pl.ds(0, 120)], acc.at[pl.ds(zoff, 120)])

    zoff5 = pl.multiple_of(s * _RPT8 + 600, 8)
    pltpu.sync_copy(rows0.at[pl.ds(0, 24)], acc.at[pl.ds(zoff5, 24)])

    @pl.when(s == _NS - 1)
    def _zero_tail():
        pltpu.sync_copy(rows0.at[pl.ds(0, _TAIL)],
                        acc.at[pl.ds(_NS * _RPT8, _TAIL)])

    plsc.subcore_barrier()

    _sidx_wait(sidx0, samA0)
    _gather(sidx0, rows0, gsem0)
    _sidx_wait(sidx1, samA1)
    _gather(sidx1, rows1, gsem1)

    @pl.loop(0, _ITERS, step=2)
    def _pair(g):
        _gwait(rows0, gsem0)

        @pl.when(g + 2 < _ITERS)
        def _pfA0():
            _sidx(g + 2, sidx0, samA0)

        _didx_wait(didx0, samB0)
        pltpu.async_copy(rows0, acc.at[didx0], ssem0, add=True)

        _gwait(rows1, gsem1)

        @pl.when(g + 3 < _ITERS)
        def _pfA1():
            _sidx(g + 3, sidx1, samA1)

        _didx_wait(didx1, samB1)
        pltpu.async_copy(rows1, acc.at[didx1], ssem1, add=True)

        pltpu.make_async_copy(rows0, acc.at[didx0], ssem0).wait()

        @pl.when(g + 2 < _ITERS)
        def _nx0():
            _didx(g + 2, didx0, samB0)
            _sidx_wait(sidx0, samA0)
            _gather(sidx0, rows0, gsem0)

        pltpu.make_async_copy(rows1, acc.at[didx1], ssem1).wait()

        @pl.when(g + 3 < _ITERS)
        def _nx1():
            _didx(g + 3, didx1, samB1)
            _sidx_wait(sidx1, samA1)
            _gather(sidx1, rows1, gsem1)

    plsc.subcore_barrier()

    doff = pl.multiple_of(s * _RPT8, 8)
    pltpu.sync_copy(acc.at[pl.ds(doff, _RPT8)],
                    out_hbm.at[c, pl.ds(doff, _RPT8)])

    @pl.when(s == _NS - 1)
    def _drain_tail():
        pltpu.sync_copy(acc.at[pl.ds(_NS * _RPT8, _TAIL)],
                        out_hbm.at[c, pl.ds(_NS * _RPT8, _TAIL)])


@functools.partial(
    pl.kernel,
    mesh=plsc.VectorSubcoreMesh(core_axis_name="c", subcore_axis_name="s"),
    out_type=jax.ShapeDtypeStruct((_NC, _N, _W), jnp.float32),
    scratch_types=[
        pltpu.VMEM((_C,), jnp.int32),
        pltpu.VMEM((_C,), jnp.int32),
        pltpu.VMEM((_C,), jnp.int32),
        pltpu.VMEM((_C,), jnp.int32),
        pltpu.VMEM((_C, _W), jnp.float32),
        pltpu.VMEM((_C, _W), jnp.float32),
        pltpu.VMEM_SHARED((_N, _W), jnp.float32),
        pltpu.SemaphoreType.DMA,
        pltpu.SemaphoreType.DMA,
        pltpu.SemaphoreType.DMA,
        pltpu.SemaphoreType.DMA,
        pltpu.SemaphoreType.DMA,
        pltpu.SemaphoreType.DMA,
        pltpu.SemaphoreType.DMA,
        pltpu.SemaphoreType.DMA,
    ],
    compiler_params=pltpu.CompilerParams(use_tc_tiling_on_sc=False),
)
def _edge_pass(w_hbm, ei_hbm, out_hbm,
               sidx0, sidx1, didx0, didx1, rows0, rows1, acc,
               gsem0, gsem1, ssem0, ssem1, samA0, samA1, samB0, samB1):
    _edge_body(w_hbm, ei_hbm, out_hbm,
               sidx0, sidx1, didx0, didx1, rows0, rows1, acc,
               gsem0, gsem1, ssem0, ssem1, samA0, samA1, samB0, samB1)


def _final_body(acc_ref, gselt_ref, out_ref):
    tot = acc_ref[0] + acc_ref[1]
    num = tot[:, 0:_HD]
    den = tot[:, _HD:_HD + _H]
    denb = jnp.dot(den, gselt_ref[...], precision=lax.Precision.HIGHEST)
    out_ref[...] = jnp.where(denb > 0, num / denb, 0.0)


def _final(acc, gselt):
    return pl.pallas_call(
        _final_body,
        out_shape=jax.ShapeDtypeStruct((_N, _HD), jnp.float32),
    )(acc, gselt)


def kernel(h_src, h_dst, attn_l, edge_index):
    del h_dst
    attn_flat = attn_l.reshape(1, _HD).astype(jnp.float32)
    gsel = (jnp.arange(_HD)[:, None] // _D == jnp.arange(_H)[None, :]
            ).astype(jnp.float32)
    gselt = gsel.T
    w = _prep(h_src, attn_flat, gsel, gselt)
    ei = edge_index.reshape(2 * _E // _C, _C)
    acc = _edge_pass(w, ei)
    return _final(acc, gselt)


# speedup vs baseline: 1.0942x; 1.0880x over previous
import functools

import jax
import jax.numpy as jnp
from jax import lax
from jax.experimental import pallas as pl
from jax.experimental.pallas import tpu as pltpu
from jax.experimental.pallas import tpu_sc as plsc

_N = 10000
_E = 320000
_H = 8
_D = 16
_NEG_SLOPE = 0.2
_HD = _H * _D
_W = 144

_NC = 2
_NS = 16
_NW = _NC * _NS
_EPT = _E // _NW
_C = 125
_ITERS = _EPT // _C
_RPT8 = 624
_TAIL = _N - _NS * _RPT8


def _prep_body(hs_ref, attn_ref, gsel_ref, gselt_ref, out_ref):
    hs = hs_ref[...]
    attn = attn_ref[...]
    gsel = gsel_ref[...]
    gselt = gselt_ref[...]
    el = jnp.dot(hs * attn, gsel, precision=lax.Precision.HIGHEST)
    el = jnp.where(el > 0, el, _NEG_SLOPE * el)
    m = jnp.max(el)
    q = jnp.exp(el - m)
    qb = jnp.dot(q, gselt, precision=lax.Precision.HIGHEST)
    w = hs * qb
    pad = jnp.zeros((hs.shape[0], _W - _HD - _H), jnp.float32)
    out_ref[...] = jnp.concatenate([w, q, pad], axis=1)


def _prep(h_src, attn_flat, gsel, gselt):
    return pl.pallas_call(
        _prep_body,
        out_shape=jax.ShapeDtypeStruct((_N, _W), jnp.float32),
    )(h_src, attn_flat, gsel, gselt)


def _edge_body(w_hbm, ei_hbm, out_hbm,
               sidx0, sidx1, didx0, didx1, rows0, rows1, acc,
               gsem0, gsem1, ssem0, ssem1, samA0, samA1, samB0, samB1):
    c = lax.axis_index("c")
    s = lax.axis_index("s")
    wid = s * _NC + c
    sbase = wid * _ITERS
    dbase = _E // _C + wid * _ITERS

    def _sidx(i, buf, sem):
        pltpu.async_copy(ei_hbm.at[sbase + i], buf, sem)

    def _sidx_wait(buf, sem):
        pltpu.make_async_copy(ei_hbm.at[sbase], buf, sem).wait()

    def _didx(i, buf, sem):
        pltpu.async_copy(ei_hbm.at[dbase + i], buf, sem)

    def _didx_wait(buf, sem):
        pltpu.make_async_copy(ei_hbm.at[dbase], buf, sem).wait()

    def _gather(sbuf, rows, gsem):
        pltpu.async_copy(w_hbm.at[sbuf], rows, gsem)

    def _gwait(rows, gsem):
        pltpu.make_async_copy(w_hbm.at[sidx0], rows, gsem).wait()

    _sidx(0, sidx0, samA0)
    _sidx(1, sidx1, samA1)
    _didx(0, didx0, samB0)
    _didx(1, didx1, samB1)

    @pl.loop(0, 120)
    def _zrow(r):
        for k in range(_W // 16):
            rows0[r, pl.ds(16 * k, 16)] = jnp.zeros((16,), jnp.float32)

    @pl.loop(0, 5)
    def _zcp(j):
        zoff = pl.multiple_of(s * _RPT8 + j * 120, 8)
        pltpu.sync_copy(rows0.at[pl.ds(0, 120)], acc.at[pl.ds(zoff, 120)])

    zoff5 = pl.multiple_of(s * _RPT8 + 600, 8)
    pltpu.sync_copy(rows0.at[pl.ds(0, 24)], acc.at[pl.ds(zoff5, 24)])

    @pl.when(s == _NS - 1)
    def _zero_tail():
        pltpu.sync_copy(rows0.at[pl.ds(0, _TAIL)],
                        acc.at[pl.ds(_NS * _RPT8, _TAIL)])

    plsc.subcore_barrier()

    _sidx_wait(sidx0, samA0)
    _gather(sidx0, rows0, gsem0)

    @pl.loop(0, _ITERS, step=2)
    def _pair(g):
        _gwait(rows0, gsem0)
        _sidx_wait(sidx1, samA1)
        _gather(sidx1, rows1, gsem1)
        _didx_wait(didx0, samB0)
        pltpu.sync_copy(rows0, acc.at[didx0], add=True)

        @pl.when(g + 2 < _ITERS)
        def _idx2():
            _sidx(g + 2, sidx0, samA0)
            _didx(g + 2, didx0, samB0)

        _gwait(rows1, gsem1)

        @pl.when(g + 2 < _ITERS)
        def _g2():
            _sidx_wait(sidx0, samA0)
            _gather(sidx0, rows0, gsem0)

        _didx_wait(didx1, samB1)
        pltpu.sync_copy(rows1, acc.at[didx1], add=True)

        @pl.when(g + 3 < _ITERS)
        def _idx3():
            _sidx(g + 3, sidx1, samA1)
            _didx(g + 3, didx1, samB1)

    plsc.subcore_barrier()

    doff = pl.multiple_of(s * _RPT8, 8)
    pltpu.sync_copy(acc.at[pl.ds(doff, _RPT8)],
                    out_hbm.at[c, pl.ds(doff, _RPT8)])

    @pl.when(s == _NS - 1)
    def _drain_tail():
        pltpu.sync_copy(acc.at[pl.ds(_NS * _RPT8, _TAIL)],
                        out_hbm.at[c, pl.ds(_NS * _RPT8, _TAIL)])


@functools.partial(
    pl.kernel,
    mesh=plsc.VectorSubcoreMesh(core_axis_name="c", subcore_axis_name="s"),
    out_type=jax.ShapeDtypeStruct((_NC, _N, _W), jnp.float32),
    scratch_types=[
        pltpu.VMEM((_C,), jnp.int32),
        pltpu.VMEM((_C,), jnp.int32),
        pltpu.VMEM((_C,), jnp.int32),
        pltpu.VMEM((_C,), jnp.int32),
        pltpu.VMEM((_C, _W), jnp.float32),
        pltpu.VMEM((_C, _W), jnp.float32),
        pltpu.VMEM_SHARED((_N, _W), jnp.float32),
        pltpu.SemaphoreType.DMA,
        pltpu.SemaphoreType.DMA,
        pltpu.SemaphoreType.DMA,
        pltpu.SemaphoreType.DMA,
        pltpu.SemaphoreType.DMA,
        pltpu.SemaphoreType.DMA,
        pltpu.SemaphoreType.DMA,
        pltpu.SemaphoreType.DMA,
    ],
    compiler_params=pltpu.CompilerParams(use_tc_tiling_on_sc=False),
)
def _edge_pass(w_hbm, ei_hbm, out_hbm,
               sidx0, sidx1, didx0, didx1, rows0, rows1, acc,
               gsem0, gsem1, ssem0, ssem1, samA0, samA1, samB0, samB1):
    _edge_body(w_hbm, ei_hbm, out_hbm,
               sidx0, sidx1, didx0, didx1, rows0, rows1, acc,
               gsem0, gsem1, ssem0, ssem1, samA0, samA1, samB0, samB1)


def _final_body(acc_ref, gselt_ref, out_ref):
    tot = acc_ref[0] + acc_ref[1]
    num = tot[:, 0:_HD]
    den = tot[:, _HD:_HD + _H]
    denb = jnp.dot(den, gselt_ref[...], precision=lax.Precision.HIGHEST)
    out_ref[...] = jnp.where(denb > 0, num / denb, 0.0)


def _final(acc, gselt):
    return pl.pallas_call(
        _final_body,
        out_shape=jax.ShapeDtypeStruct((_N, _HD), jnp.float32),
    )(acc, gselt)


def kernel(h_src, h_dst, attn_l, edge_index):
    del h_dst
    attn_flat = attn_l.reshape(1, _HD).astype(jnp.float32)
    gsel = (jnp.arange(_HD)[:, None] // _D == jnp.arange(_H)[None, :]
            ).astype(jnp.float32)
    gselt = gsel.T
    w = _prep(h_src, attn_flat, gsel, gselt)
    ei = edge_index.reshape(2 * _E // _C, _C)
    acc = _edge_pass(w, ei)
    return _final(acc, gselt)


# speedup vs baseline: 1.1757x; 1.0745x over previous
import functools

import jax
import jax.numpy as jnp
from jax import lax
from jax.experimental import pallas as pl
from jax.experimental.pallas import tpu as pltpu
from jax.experimental.pallas import tpu_sc as plsc

_N = 10000
_E = 320000
_H = 8
_D = 16
_NEG_SLOPE = 0.2
_HD = _H * _D
_W = 144

_NC = 2
_NS = 16
_NW = _NC * _NS
_EPT = _E // _NW
_C = 80
_ITERS = _EPT // _C
_RPT8 = 624
_TAIL = _N - _NS * _RPT8


def _prep_body(hs_ref, attn_ref, gsel_ref, gselt_ref, out_ref):
    hs = hs_ref[...]
    attn = attn_ref[...]
    gsel = gsel_ref[...]
    gselt = gselt_ref[...]
    el = jnp.dot(hs * attn, gsel, precision=lax.Precision.HIGHEST)
    el = jnp.where(el > 0, el, _NEG_SLOPE * el)
    m = jnp.max(el)
    q = jnp.exp(el - m)
    qb = jnp.dot(q, gselt, precision=lax.Precision.HIGHEST)
    w = hs * qb
    pad = jnp.zeros((hs.shape[0], _W - _HD - _H), jnp.float32)
    out_ref[...] = jnp.concatenate([w, q, pad], axis=1)


def _prep(h_src, attn_flat, gsel, gselt):
    return pl.pallas_call(
        _prep_body,
        out_shape=jax.ShapeDtypeStruct((_N, _W), jnp.float32),
    )(h_src, attn_flat, gsel, gselt)


def _edge_body(w_hbm, ei_hbm, out_hbm,
               sidx0, sidx1, sidx2, didx0, didx1, didx2,
               rows0, rows1, rows2, acc,
               gsem0, gsem1, gsem2, samA0, samA1, samA2,
               samB0, samB1, samB2):
    c = lax.axis_index("c")
    s = lax.axis_index("s")
    wid = s * _NC + c
    sbase = wid * _ITERS
    dbase = _E // _C + wid * _ITERS

    sidx = (sidx0, sidx1, sidx2)
    didx = (didx0, didx1, didx2)
    rows = (rows0, rows1, rows2)
    gsem = (gsem0, gsem1, gsem2)
    samA = (samA0, samA1, samA2)
    samB = (samB0, samB1, samB2)

    def _sidx_ld(i, b):
        pltpu.async_copy(ei_hbm.at[sbase + i], sidx[b], samA[b])

    def _sidx_wait(b):
        pltpu.make_async_copy(ei_hbm.at[sbase], sidx[b], samA[b]).wait()

    def _didx_ld(i, b):
        pltpu.async_copy(ei_hbm.at[dbase + i], didx[b], samB[b])

    def _didx_wait(b):
        pltpu.make_async_copy(ei_hbm.at[dbase], didx[b], samB[b]).wait()

    def _gather(b):
        pltpu.async_copy(w_hbm.at[sidx[b]], rows[b], gsem[b])

    def _gwait(b):
        pltpu.make_async_copy(w_hbm.at[sidx0], rows[b], gsem[b]).wait()

    for b in range(3):
        _sidx_ld(b, b)
        _didx_ld(b, b)

    @pl.loop(0, _C)
    def _zrow(r):
        for k in range(_W // 16):
            rows0[r, pl.ds(16 * k, 16)] = jnp.zeros((16,), jnp.float32)

    @pl.loop(0, 7)
    def _zcp(j):
        zoff = pl.multiple_of(s * _RPT8 + j * _C, 8)
        pltpu.sync_copy(rows0, acc.at[pl.ds(zoff, _C)])

    zoff7 = pl.multiple_of(s * _RPT8 + 7 * _C, 8)
    pltpu.sync_copy(rows0.at[pl.ds(0, _RPT8 - 7 * _C)],
                    acc.at[pl.ds(zoff7, _RPT8 - 7 * _C)])

    @pl.when(s == _NS - 1)
    def _zero_tail():
        pltpu.sync_copy(rows0.at[pl.ds(0, _TAIL)],
                        acc.at[pl.ds(_NS * _RPT8, _TAIL)])

    plsc.subcore_barrier()

    _sidx_wait(0)
    _gather(0)
    _sidx_wait(1)
    _gather(1)

    def _slot(i, b):
        b2 = (b + 2) % 3
        _gwait(b)
        _didx_wait(b)
        pltpu.sync_copy(rows[b], acc.at[didx[b]], add=True)

        @pl.when(i + 3 < _ITERS)
        def _pf():
            _sidx_ld(i + 3, b)
            _didx_ld(i + 3, b)

        @pl.when(i + 2 < _ITERS)
        def _g2():
            _sidx_wait(b2)
            _gather(b2)

    @pl.loop(0, _ITERS - 2, step=3)
    def _round(g):
        _slot(g, 0)
        _slot(g + 1, 1)
        _slot(g + 2, 2)

    _gwait(0)
    _didx_wait(0)
    pltpu.sync_copy(rows0, acc.at[didx0], add=True)
    _gwait(1)
    _didx_wait(1)
    pltpu.sync_copy(rows1, acc.at[didx1], add=True)

    plsc.subcore_barrier()

    doff = pl.multiple_of(s * _RPT8, 8)
    pltpu.sync_copy(acc.at[pl.ds(doff, _RPT8)],
                    out_hbm.at[c, pl.ds(doff, _RPT8)])

    @pl.when(s == _NS - 1)
    def _drain_tail():
        pltpu.sync_copy(acc.at[pl.ds(_NS * _RPT8, _TAIL)],
                        out_hbm.at[c, pl.ds(_NS * _RPT8, _TAIL)])


@functools.partial(
    pl.kernel,
    mesh=plsc.VectorSubcoreMesh(core_axis_name="c", subcore_axis_name="s"),
    out_type=jax.ShapeDtypeStruct((_NC, _N, _W), jnp.float32),
    scratch_types=[
        pltpu.VMEM((_C,), jnp.int32),
        pltpu.VMEM((_C,), jnp.int32),
        pltpu.VMEM((_C,), jnp.int32),
        pltpu.VMEM((_C,), jnp.int32),
        pltpu.VMEM((_C,), jnp.int32),
        pltpu.VMEM((_C,), jnp.int32),
        pltpu.VMEM((_C, _W), jnp.float32),
        pltpu.VMEM((_C, _W), jnp.float32),
        pltpu.VMEM((_C, _W), jnp.float32),
        pltpu.VMEM_SHARED((_N, _W), jnp.float32),
        pltpu.SemaphoreType.DMA,
        pltpu.SemaphoreType.DMA,
        pltpu.SemaphoreType.DMA,
        pltpu.SemaphoreType.DMA,
        pltpu.SemaphoreType.DMA,
        pltpu.SemaphoreType.DMA,
        pltpu.SemaphoreType.DMA,
        pltpu.SemaphoreType.DMA,
        pltpu.SemaphoreType.DMA,
    ],
    compiler_params=pltpu.CompilerParams(use_tc_tiling_on_sc=False),
)
def _edge_pass(w_hbm, ei_hbm, out_hbm,
               sidx0, sidx1, sidx2, didx0, didx1, didx2,
               rows0, rows1, rows2, acc,
               gsem0, gsem1, gsem2, samA0, samA1, samA2,
               samB0, samB1, samB2):
    _edge_body(w_hbm, ei_hbm, out_hbm,
               sidx0, sidx1, sidx2, didx0, didx1, didx2,
               rows0, rows1, rows2, acc,
               gsem0, gsem1, gsem2, samA0, samA1, samA2,
               samB0, samB1, samB2)


def _final_body(acc_ref, gselt_ref, out_ref):
    tot = acc_ref[0] + acc_ref[1]
    num = tot[:, 0:_HD]
    den = tot[:, _HD:_HD + _H]
    denb = jnp.dot(den, gselt_ref[...], precision=lax.Precision.HIGHEST)
    out_ref[...] = jnp.where(denb > 0, num / denb, 0.0)


def _final(acc, gselt):
    return pl.pallas_call(
        _final_body,
        out_shape=jax.ShapeDtypeStruct((_N, _HD), jnp.float32),
    )(acc, gselt)


def kernel(h_src, h_dst, attn_l, edge_index):
    del h_dst
    attn_flat = attn_l.reshape(1, _HD).astype(jnp.float32)
    gsel = (jnp.arange(_HD)[:, None] // _D == jnp.arange(_H)[None, :]
            ).astype(jnp.float32)
    gselt = gsel.T
    w = _prep(h_src, attn_flat, gsel, gselt)
    ei = edge_index.reshape(2 * _E // _C, _C)
    acc = _edge_pass(w, ei)
    return _final(acc, gselt)
